# Initial kernel scaffold; baseline (speedup 1.0000x reference)
import functools

import jax
import jax.numpy as jnp
from jax import lax
from jax.experimental import pallas as pl
from jax.experimental.pallas import tpu as pltpu
from jax.experimental.pallas import tpu_sc as plsc

N_VOCAB = 100000
EMBED_DIM = 64
OFFSET = 20
BATCH = 4096

_info = plsc.get_sparse_core_info()
NC, NS, L = _info.num_cores, _info.num_subcores, _info.num_lanes
NW = NC * NS
BAGS_PER_W = BATCH // NW
IDX_PER_W = BAGS_PER_W * OFFSET
IDX_ROWS = IDX_PER_W // 128
CHUNK_BAGS = 64
CHUNK_ROWS = CHUNK_BAGS * OFFSET
CHUNK_IDX_ROWS = CHUNK_ROWS // 128
N_CHUNKS = BAGS_PER_W // CHUNK_BAGS
VREGS_PER_ROW = EMBED_DIM // L


def _bag_kernel(idx_hbm, weight_hbm, out_hbm, idx_v, rows_v, out_v, sem):
    wid = lax.axis_index("s") * NC + lax.axis_index("c")
    pltpu.sync_copy(idx_hbm.at[wid], idx_v)

    for c in range(N_CHUNKS):
        copies = [
            pltpu.async_copy(
                weight_hbm.at[idx_v.at[c * CHUNK_IDX_ROWS + j]],
                rows_v.at[pl.ds(j * 128, 128)],
                sem,
            )
            for j in range(CHUNK_IDX_ROWS)
        ]
        for cp in copies:
            cp.wait()

        def reduce_bag(b, carry):
            base = b * OFFSET
            for v in range(VREGS_PER_ROW):
                sl = pl.ds(v * L, L)
                acc = rows_v[base, sl]
                for r in range(1, OFFSET):
                    acc = acc + rows_v[base + r, sl]
                out_v[b, sl] = acc
            return carry

        lax.fori_loop(0, CHUNK_BAGS, reduce_bag, 0)

        pltpu.sync_copy(
            out_v,
            out_hbm.at[pl.ds(wid * BAGS_PER_W + c * CHUNK_BAGS, CHUNK_BAGS)],
        )


@jax.jit
def _bag(idx, weight):
    mesh = plsc.VectorSubcoreMesh(core_axis_name="c", subcore_axis_name="s")
    return pl.kernel(
        _bag_kernel,
        mesh=mesh,
        out_type=jax.ShapeDtypeStruct((BATCH, EMBED_DIM), jnp.float32),
        scratch_types=[
            pltpu.VMEM((IDX_ROWS, 128), jnp.int32),
            pltpu.VMEM((CHUNK_ROWS, EMBED_DIM), jnp.float32),
            pltpu.VMEM((CHUNK_BAGS, EMBED_DIM), jnp.float32),
            pltpu.SemaphoreType.DMA,
        ],
    )(idx, weight)


def kernel(sparse_index_group_batch, sparse_offset_group_batch, weight):
    del sparse_offset_group_batch
    idx = sparse_index_group_batch.astype(jnp.int32).reshape(NW, IDX_ROWS, 128)
    return _bag(idx, weight)


# speedup vs baseline: 1.4056x; 1.4056x over previous
import functools

import jax
import jax.numpy as jnp
from jax import lax
from jax.experimental import pallas as pl
from jax.experimental.pallas import tpu as pltpu
from jax.experimental.pallas import tpu_sc as plsc

N_VOCAB = 100000
EMBED_DIM = 64
OFFSET = 20
BATCH = 4096

_info = plsc.get_sparse_core_info()
NC, NS, L = _info.num_cores, _info.num_subcores, _info.num_lanes
NW = NC * NS
BAGS_PER_W = BATCH // NW
IDX_PER_W = BAGS_PER_W * OFFSET
IDX_ROWS = IDX_PER_W // 128
CHUNK_BAGS = 64
CHUNK_ROWS = CHUNK_BAGS * OFFSET
CHUNK_IDX_ROWS = CHUNK_ROWS // 128
N_CHUNKS = BAGS_PER_W // CHUNK_BAGS
VREGS_PER_ROW = EMBED_DIM // L


def _bag_kernel(idx_hbm, weight_hbm, out_hbm, idx_v, rows_v, out_v, sem):
    wid = lax.axis_index("s") * NC + lax.axis_index("c")
    pltpu.sync_copy(idx_hbm.at[wid], idx_v)

    for c in range(N_CHUNKS):
        copies = [
            pltpu.async_copy(
                weight_hbm.at[idx_v.at[c * CHUNK_IDX_ROWS + j]],
                rows_v.at[pl.ds(j * 128, 128)],
                sem,
            )
            for j in range(CHUNK_IDX_ROWS)
        ]
        for cp in copies:
            cp.wait()

        def reduce_bag(b, carry):
            base = b * OFFSET
            for v in range(VREGS_PER_ROW):
                sl = pl.ds(v * L, L)
                acc = rows_v[base, sl]
                for r in range(1, OFFSET):
                    acc = acc + rows_v[base + r, sl]
                out_v[b, sl] = acc
            return carry

        lax.fori_loop(0, CHUNK_BAGS, reduce_bag, 0)

        pltpu.sync_copy(
            out_v,
            out_hbm.at[pl.ds(wid * BAGS_PER_W + c * CHUNK_BAGS, CHUNK_BAGS)],
        )


@jax.jit
def _bag(idx, weight):
    mesh = plsc.VectorSubcoreMesh(core_axis_name="c", subcore_axis_name="s")
    return pl.kernel(
        _bag_kernel,
        mesh=mesh,
        compiler_params=pltpu.CompilerParams(use_tc_tiling_on_sc=False),
        out_type=jax.ShapeDtypeStruct((BATCH, EMBED_DIM), jnp.float32),
        scratch_types=[
            pltpu.VMEM((IDX_ROWS, 128), jnp.int32),
            pltpu.VMEM((CHUNK_ROWS, EMBED_DIM), jnp.float32),
            pltpu.VMEM((CHUNK_BAGS, EMBED_DIM), jnp.float32),
            pltpu.SemaphoreType.DMA,
        ],
    )(idx, weight)


def kernel(sparse_index_group_batch, sparse_offset_group_batch, weight):
    del sparse_offset_group_batch
    idx = sparse_index_group_batch.astype(jnp.int32).reshape(NW, IDX_ROWS, 128)
    return _bag(idx, weight)


# speedup vs baseline: 1.4092x; 1.0026x over previous
import functools

import jax
import jax.numpy as jnp
from jax import lax
from jax.experimental import pallas as pl
from jax.experimental.pallas import tpu as pltpu
from jax.experimental.pallas import tpu_sc as plsc

N_VOCAB = 100000
EMBED_DIM = 64
OFFSET = 20
BATCH = 4096

_info = plsc.get_sparse_core_info()
NC, NS, L = _info.num_cores, _info.num_subcores, _info.num_lanes
NW = NC * NS
BAGS_PER_W = BATCH // NW
IDX_PER_W = BAGS_PER_W * OFFSET
IDX_ROWS = IDX_PER_W // 128
CHUNK_BAGS = 64
CHUNK_ROWS = CHUNK_BAGS * OFFSET
CHUNK_IDX_ROWS = CHUNK_ROWS // 128
N_CHUNKS = BAGS_PER_W // CHUNK_BAGS
VREGS_PER_ROW = EMBED_DIM // L


def _bag_kernel(idx_hbm, weight_hbm, out_hbm, idx_v, rows_v, out_v, sem):
    wid = lax.axis_index("s") * NC + lax.axis_index("c")
    pltpu.sync_copy(idx_hbm.at[pl.ds(wid * IDX_PER_W, IDX_PER_W)], idx_v)

    for c in range(N_CHUNKS):
        copies = [
            pltpu.async_copy(
                weight_hbm.at[idx_v.at[pl.ds((c * CHUNK_IDX_ROWS + j) * 128, 128)]],
                rows_v.at[pl.ds(j * 128, 128)],
                sem,
            )
            for j in range(CHUNK_IDX_ROWS)
        ]
        for cp in copies:
            cp.wait()

        def reduce_bag(b, carry):
            base = b * OFFSET
            for v in range(VREGS_PER_ROW):
                sl = pl.ds(v * L, L)
                acc = rows_v[base, sl]
                for r in range(1, OFFSET):
                    acc = acc + rows_v[base + r, sl]
                out_v[b, sl] = acc
            return carry

        lax.fori_loop(0, CHUNK_BAGS, reduce_bag, 0)

        pltpu.sync_copy(
            out_v,
            out_hbm.at[pl.ds(wid * BAGS_PER_W + c * CHUNK_BAGS, CHUNK_BAGS)],
        )


@jax.jit
def _bag(idx, weight):
    mesh = plsc.VectorSubcoreMesh(core_axis_name="c", subcore_axis_name="s")
    return pl.kernel(
        _bag_kernel,
        mesh=mesh,
        compiler_params=pltpu.CompilerParams(use_tc_tiling_on_sc=False),
        out_type=jax.ShapeDtypeStruct((BATCH, EMBED_DIM), jnp.float32),
        scratch_types=[
            pltpu.VMEM((IDX_PER_W,), jnp.int32),
            pltpu.VMEM((CHUNK_ROWS, EMBED_DIM), jnp.float32),
            pltpu.VMEM((CHUNK_BAGS, EMBED_DIM), jnp.float32),
            pltpu.SemaphoreType.DMA,
        ],
    )(idx, weight)


def kernel(sparse_index_group_batch, sparse_offset_group_batch, weight):
    del sparse_offset_group_batch
    idx = sparse_index_group_batch.astype(jnp.int32)
    return _bag(idx, weight)
